# initial kernel scaffold (unmeasured)
import jax
import jax.numpy as jnp
from jax import lax
from jax.experimental import pallas as pl
from jax.experimental.pallas import tpu as pltpu

N_DEV = 4


def kernel(x, w_mat, scale_x, scale_w):
    m_per, k = x.shape
    _, n_per = w_mat.shape
    half = m_per // 2

    def body(x_ref, w_ref, sx_ref, sw_ref, out_ref,
             buf_a, buf_b, w16, send_a, recv_a, send_b, recv_b):
        my = lax.axis_index("i")
        left = lax.rem(my + N_DEV - 1, N_DEV)
        right = lax.rem(my + 1, N_DEV)

        barrier = pltpu.get_barrier_semaphore()
        for nbr in (left, right):
            pl.semaphore_signal(
                barrier, inc=1,
                device_id=(nbr,), device_id_type=pl.DeviceIdType.MESH,
            )
        pl.semaphore_wait(barrier, 2)

        buf_a[0] = x_ref[0:half, :].astype(jnp.float8_e4m3fn)
        buf_b[0] = x_ref[half:m_per, :].astype(jnp.float8_e4m3fn)
        w16[...] = w_ref[...].astype(jnp.bfloat16)

        scale = sx_ref[0, 0] * sw_ref[0, 0]

        def compute_slot(s):
            origin_a = lax.rem(my - s + N_DEV, N_DEV)
            origin_b = lax.rem(my + s, N_DEV)
            acc_a = jnp.dot(buf_a[s].astype(jnp.bfloat16), w16[...],
                            preferred_element_type=jnp.float32)
            out_ref[pl.ds(origin_a * m_per, half), :] = (
                jnp.maximum(acc_a * scale, 0.0))
            acc_b = jnp.dot(buf_b[s].astype(jnp.bfloat16), w16[...],
                            preferred_element_type=jnp.float32)
            out_ref[pl.ds(origin_b * m_per + half, half), :] = (
                jnp.maximum(acc_b * scale, 0.0))

        for h in range(N_DEV - 1):
            rdma_a = pltpu.make_async_remote_copy(
                src_ref=buf_a.at[h],
                dst_ref=buf_a.at[h + 1],
                send_sem=send_a.at[h],
                recv_sem=recv_a.at[h],
                device_id=(right,),
                device_id_type=pl.DeviceIdType.MESH,
            )
            rdma_b = pltpu.make_async_remote_copy(
                src_ref=buf_b.at[h],
                dst_ref=buf_b.at[h + 1],
                send_sem=send_b.at[h],
                recv_sem=recv_b.at[h],
                device_id=(left,),
                device_id_type=pl.DeviceIdType.MESH,
            )
            rdma_a.start()
            rdma_b.start()
            compute_slot(h)
            rdma_a.wait()
            rdma_b.wait()
        compute_slot(N_DEV - 1)

    return pl.pallas_call(
        body,
        out_shape=jax.ShapeDtypeStruct((N_DEV * m_per, n_per), jnp.float32),
        in_specs=[
            pl.BlockSpec(memory_space=pltpu.VMEM),
            pl.BlockSpec(memory_space=pltpu.VMEM),
            pl.BlockSpec(memory_space=pltpu.SMEM),
            pl.BlockSpec(memory_space=pltpu.SMEM),
        ],
        out_specs=pl.BlockSpec(memory_space=pltpu.VMEM),
        scratch_shapes=[
            pltpu.VMEM((N_DEV, half, k), jnp.float8_e4m3fn),
            pltpu.VMEM((N_DEV, half, k), jnp.float8_e4m3fn),
            pltpu.VMEM((k, n_per), jnp.bfloat16),
            pltpu.SemaphoreType.DMA((N_DEV - 1,)),
            pltpu.SemaphoreType.DMA((N_DEV - 1,)),
            pltpu.SemaphoreType.DMA((N_DEV - 1,)),
            pltpu.SemaphoreType.DMA((N_DEV - 1,)),
        ],
        compiler_params=pltpu.CompilerParams(collective_id=0),
    )(x, w_mat, scale_x.reshape(1, 1), scale_w.reshape(1, 1))


# baseline (device time: 105904 ns/iter reference)
import jax
import jax.numpy as jnp
from jax import lax
from jax.experimental import pallas as pl
from jax.experimental.pallas import tpu as pltpu

N_DEV = 4


def kernel(x, w_mat, scale_x, scale_w):
    m_per, k = x.shape
    _, n_per = w_mat.shape
    half = m_per // 2

    xq = x.astype(jnp.float8_e4m3fn)
    wq = w_mat.astype(jnp.bfloat16)

    def body(x_ref, w_ref, sx_ref, sw_ref, out_ref,
             buf_a, buf_b, send_a, recv_a, send_b, recv_b):
        my = lax.axis_index("i")
        left = lax.rem(my + N_DEV - 1, N_DEV)
        right = lax.rem(my + 1, N_DEV)

        barrier = pltpu.get_barrier_semaphore()
        for nbr in (left, right):
            pl.semaphore_signal(
                barrier, inc=1,
                device_id=(nbr,), device_id_type=pl.DeviceIdType.MESH,
            )
        pl.semaphore_wait(barrier, 2)

        buf_a[0] = x_ref[0:half, :]
        buf_b[0] = x_ref[half:m_per, :]

        scale = sx_ref[0, 0] * sw_ref[0, 0]

        def compute_slot(s):
            origin_a = lax.rem(my - s + N_DEV, N_DEV)
            origin_b = lax.rem(my + s, N_DEV)
            acc_a = jnp.dot(buf_a[s].astype(jnp.bfloat16), w_ref[...],
                            preferred_element_type=jnp.float32)
            out_ref[pl.ds(origin_a * m_per, half), :] = (
                jnp.maximum(acc_a * scale, 0.0))
            acc_b = jnp.dot(buf_b[s].astype(jnp.bfloat16), w_ref[...],
                            preferred_element_type=jnp.float32)
            out_ref[pl.ds(origin_b * m_per + half, half), :] = (
                jnp.maximum(acc_b * scale, 0.0))

        for h in range(N_DEV - 1):
            rdma_a = pltpu.make_async_remote_copy(
                src_ref=buf_a.at[h],
                dst_ref=buf_a.at[h + 1],
                send_sem=send_a.at[h],
                recv_sem=recv_a.at[h],
                device_id=(right,),
                device_id_type=pl.DeviceIdType.MESH,
            )
            rdma_b = pltpu.make_async_remote_copy(
                src_ref=buf_b.at[h],
                dst_ref=buf_b.at[h + 1],
                send_sem=send_b.at[h],
                recv_sem=recv_b.at[h],
                device_id=(left,),
                device_id_type=pl.DeviceIdType.MESH,
            )
            rdma_a.start()
            rdma_b.start()
            compute_slot(h)
            rdma_a.wait()
            rdma_b.wait()
        compute_slot(N_DEV - 1)

    return pl.pallas_call(
        body,
        out_shape=jax.ShapeDtypeStruct((N_DEV * m_per, n_per), jnp.float32),
        in_specs=[
            pl.BlockSpec(memory_space=pltpu.VMEM),
            pl.BlockSpec(memory_space=pltpu.VMEM),
            pl.BlockSpec(memory_space=pltpu.SMEM),
            pl.BlockSpec(memory_space=pltpu.SMEM),
        ],
        out_specs=pl.BlockSpec(memory_space=pltpu.VMEM),
        scratch_shapes=[
            pltpu.VMEM((N_DEV, half, k), jnp.float8_e4m3fn),
            pltpu.VMEM((N_DEV, half, k), jnp.float8_e4m3fn),
            pltpu.SemaphoreType.DMA((N_DEV - 1,)),
            pltpu.SemaphoreType.DMA((N_DEV - 1,)),
            pltpu.SemaphoreType.DMA((N_DEV - 1,)),
            pltpu.SemaphoreType.DMA((N_DEV - 1,)),
        ],
        compiler_params=pltpu.CompilerParams(
            collective_id=0,
            vmem_limit_bytes=100 * 1024 * 1024,
        ),
    )(xq, wq, scale_x.reshape(1, 1), scale_w.reshape(1, 1))


# device time: 102012 ns/iter; 1.0382x vs baseline; 1.0382x over previous
import jax
import jax.numpy as jnp
from jax import lax
from jax.experimental import pallas as pl
from jax.experimental.pallas import tpu as pltpu

N_DEV = 4


def kernel(x, w_mat, scale_x, scale_w):
    m_per, k = x.shape
    _, n_per = w_mat.shape
    half = m_per // 2

    wq = w_mat.astype(jnp.bfloat16)

    def body(x_ref, w_ref, sx_ref, sw_ref, out_ref,
             buf_a, buf_b, send_a, recv_a, send_b, recv_b):
        my = lax.axis_index("i")
        left = lax.rem(my + N_DEV - 1, N_DEV)
        right = lax.rem(my + 1, N_DEV)

        barrier = pltpu.get_barrier_semaphore()
        for nbr in (left, right):
            pl.semaphore_signal(
                barrier, inc=1,
                device_id=(nbr,), device_id_type=pl.DeviceIdType.MESH,
            )
        pl.semaphore_wait(barrier, 2)

        buf_a[0] = x_ref[0:half, :].astype(jnp.float8_e4m3fn)
        buf_b[0] = x_ref[half:m_per, :].astype(jnp.float8_e4m3fn)

        scale = sx_ref[0, 0] * sw_ref[0, 0]

        def compute_slot(s):
            origin_a = lax.rem(my - s + N_DEV, N_DEV)
            origin_b = lax.rem(my + s, N_DEV)
            acc_a = jnp.dot(buf_a[s].astype(jnp.bfloat16), w_ref[...],
                            preferred_element_type=jnp.float32)
            out_ref[pl.ds(origin_a * m_per, half), :] = (
                jnp.maximum(acc_a * scale, 0.0))
            acc_b = jnp.dot(buf_b[s].astype(jnp.bfloat16), w_ref[...],
                            preferred_element_type=jnp.float32)
            out_ref[pl.ds(origin_b * m_per + half, half), :] = (
                jnp.maximum(acc_b * scale, 0.0))

        for h in range(N_DEV - 1):
            rdma_a = pltpu.make_async_remote_copy(
                src_ref=buf_a.at[h],
                dst_ref=buf_a.at[h + 1],
                send_sem=send_a.at[h],
                recv_sem=recv_a.at[h],
                device_id=(right,),
                device_id_type=pl.DeviceIdType.MESH,
            )
            rdma_b = pltpu.make_async_remote_copy(
                src_ref=buf_b.at[h],
                dst_ref=buf_b.at[h + 1],
                send_sem=send_b.at[h],
                recv_sem=recv_b.at[h],
                device_id=(left,),
                device_id_type=pl.DeviceIdType.MESH,
            )
            rdma_a.start()
            rdma_b.start()
            compute_slot(h)
            rdma_a.wait()
            rdma_b.wait()
        compute_slot(N_DEV - 1)

    return pl.pallas_call(
        body,
        out_shape=jax.ShapeDtypeStruct((N_DEV * m_per, n_per), jnp.float32),
        in_specs=[
            pl.BlockSpec(memory_space=pltpu.VMEM),
            pl.BlockSpec(memory_space=pltpu.VMEM),
            pl.BlockSpec(memory_space=pltpu.SMEM),
            pl.BlockSpec(memory_space=pltpu.SMEM),
        ],
        out_specs=pl.BlockSpec(memory_space=pltpu.VMEM),
        scratch_shapes=[
            pltpu.VMEM((N_DEV, half, k), jnp.float8_e4m3fn),
            pltpu.VMEM((N_DEV, half, k), jnp.float8_e4m3fn),
            pltpu.SemaphoreType.DMA((N_DEV - 1,)),
            pltpu.SemaphoreType.DMA((N_DEV - 1,)),
            pltpu.SemaphoreType.DMA((N_DEV - 1,)),
            pltpu.SemaphoreType.DMA((N_DEV - 1,)),
        ],
        compiler_params=pltpu.CompilerParams(
            collective_id=0,
            vmem_limit_bytes=100 * 1024 * 1024,
        ),
    )(x, wq, scale_x.reshape(1, 1), scale_w.reshape(1, 1))
